# Initial kernel scaffold; baseline (speedup 1.0000x reference)
import jax
import jax.numpy as jnp
from jax.experimental import pallas as pl


def kernel(x, tables, W1, b1, g1, be1, W2, b2, g2, be2, W3, b3, g3, be3, Wf, bf):
    raise NotImplementedError("write your pallas kernel here")


# speedup vs baseline: 7.2069x; 7.2069x over previous
import functools

import jax
import jax.numpy as jnp
from jax import lax
from jax.experimental import pallas as pl
from jax.experimental.pallas import tpu as pltpu
from jax.experimental.pallas import tpu_sc as plsc

F = 26
V = 100000
E = 32
B = 16384
EPS = 1e-5

NC, NS, L = 2, 16, 16
NW = NC * NS
CHUNK = 512
SUB = 128
NSUB = CHUNK // SUB
CPF = B // CHUNK
NTASK = F * CPF


def _sc_gather(x3, table_flat):
    mesh = plsc.VectorSubcoreMesh(
        core_axis_name="c", subcore_axis_name="s", num_cores=NC, num_subcores=NS
    )

    @functools.partial(
        pl.kernel,
        out_type=jax.ShapeDtypeStruct((NTASK, NSUB, SUB, E), jnp.float32),
        mesh=mesh,
        scratch_types=[
            pltpu.VMEM((NSUB, SUB), jnp.int32),
            pltpu.VMEM((NSUB, SUB, E), jnp.float32),
            pltpu.SemaphoreType.DMA,
        ],
        compiler_params=pltpu.CompilerParams(use_tc_tiling_on_sc=False),
    )
    def gather_kernel(x_hbm, tab_hbm, out_hbm, idx_v, rows_v, sem):
        wid = lax.axis_index("s") * NC + lax.axis_index("c")

        def task(j, carry):
            t = j * NW + wid
            pltpu.sync_copy(x_hbm.at[t], idx_v)
            off = j * V

            def add_off(i, c):
                k = i // (SUB // L)
                s = (i % (SUB // L)) * L
                idx_v[k, pl.ds(s, L)] = idx_v[k, pl.ds(s, L)] + off
                return c

            lax.fori_loop(0, NSUB * (SUB // L), add_off, 0)
            cps = [
                pltpu.async_copy(tab_hbm.at[idx_v.at[k]], rows_v.at[k], sem)
                for k in range(NSUB)
            ]
            for cp in cps:
                cp.wait()
            pltpu.sync_copy(rows_v, out_hbm.at[t])
            return carry

        lax.fori_loop(0, F, task, 0)

    return gather_kernel(x3, table_flat)


BB = 512
NBLK = B // BB


def _ln_relu(h, g, b):
    m = jnp.mean(h, axis=-1, keepdims=True)
    v = jnp.mean((h - m) ** 2, axis=-1, keepdims=True)
    return jnp.maximum((h - m) / jnp.sqrt(v + EPS) * g + b, 0.0)


def _mlp_body(emb_ref, w1_ref, b1_ref, g1_ref, be1_ref, w2_ref, b2_ref, g2_ref,
              be2_ref, w3_ref, b3_ref, g3_ref, be3_ref, wf_ref, bf_ref, out_ref):
    acc = jnp.zeros((BB, 64), jnp.float32)
    for f in range(F):
        acc = acc + jnp.dot(emb_ref[f], w1_ref[f],
                            preferred_element_type=jnp.float32)
    h = _ln_relu(acc + b1_ref[...], g1_ref[...], be1_ref[...])
    h = _ln_relu(jnp.dot(h, w2_ref[...], preferred_element_type=jnp.float32)
                 + b2_ref[...], g2_ref[...], be2_ref[...])
    h = _ln_relu(jnp.dot(h, w3_ref[...], preferred_element_type=jnp.float32)
                 + b3_ref[...], g3_ref[...], be3_ref[...])
    logits = jnp.sum(h * wf_ref[...], axis=-1) + bf_ref[0, 0]
    out_ref[0, 0, :] = 1.0 / (1.0 + jnp.exp(-logits))


def _tc_mlp(emb, W1, b1, g1, be1, W2, b2, g2, be2, W3, b3, g3, be3, Wf, bf):
    w1 = W1.reshape(F, E, 64)
    row = lambda a: a.reshape(1, -1)
    full = lambda s: pl.BlockSpec(s, lambda i: (0,) * len(s))
    out = pl.pallas_call(
        _mlp_body,
        grid=(NBLK,),
        in_specs=[
            pl.BlockSpec((F, BB, E), lambda i: (0, i, 0)),
            full((F, E, 64)),
            full((1, 64)), full((1, 64)), full((1, 64)),
            full((64, 32)), full((1, 32)), full((1, 32)), full((1, 32)),
            full((32, 16)), full((1, 16)), full((1, 16)), full((1, 16)),
            full((1, 16)), full((1, 1)),
        ],
        out_specs=pl.BlockSpec((1, 1, BB), lambda i: (i, 0, 0)),
        out_shape=jax.ShapeDtypeStruct((NBLK, 1, BB), jnp.float32),
    )(emb, w1, row(b1), row(g1), row(be1), W2, row(b2), row(g2), row(be2),
      W3, row(b3), row(g3), row(be3), Wf.reshape(1, 16), bf.reshape(1, 1))
    return out.reshape(B)


def kernel(x, tables, W1, b1, g1, be1, W2, b2, g2, be2, W3, b3, g3, be3, Wf, bf):
    x3 = x.T.reshape(NTASK, NSUB, SUB)
    table_flat = tables.reshape(F * V, E)
    rows = _sc_gather(x3, table_flat)
    emb = rows.reshape(F, B, E)
    return _tc_mlp(emb, W1, b1, g1, be1, W2, b2, g2, be2, W3, b3, g3, be3, Wf, bf)


# speedup vs baseline: 27.4625x; 3.8106x over previous
import functools

import jax
import jax.numpy as jnp
from jax import lax
from jax.experimental import pallas as pl
from jax.experimental.pallas import tpu as pltpu
from jax.experimental.pallas import tpu_sc as plsc

F = 26
V = 100000
E = 32
B = 16384
EPS = 1e-5

NC, NS, L = 2, 16, 16
NW = NC * NS
HALF = B // 2
UNROLL = 4


OROWS = 64


def _sc_gather(xT, tables_t):
    mesh = plsc.VectorSubcoreMesh(
        core_axis_name="c", subcore_axis_name="s", num_cores=NC, num_subcores=NS
    )

    @functools.partial(
        pl.kernel,
        out_type=jax.ShapeDtypeStruct((F * E, 128, 128), jnp.float32),
        mesh=mesh,
        scratch_types=[
            pltpu.VMEM((V,), jnp.float32),
            pltpu.VMEM((B,), jnp.int32),
            pltpu.VMEM((OROWS, 128), jnp.float32),
            pltpu.SemaphoreType.DMA,
            pltpu.SemaphoreType.DMA,
        ],
        compiler_params=pltpu.CompilerParams(
            use_tc_tiling_on_sc=True, needs_layout_passes=False
        ),
    )
    def gather_kernel(x_hbm, tab_hbm, out_hbm, vec_v, idx_v, out_v, vsem, isem):
        e = lax.axis_index("s") * NC + lax.axis_index("c")

        def task(f, carry):
            vcp = pltpu.async_copy(tab_hbm.at[f, e], vec_v, vsem)
            icp = pltpu.async_copy(x_hbm.at[f], idx_v, isem)
            vcp.wait()
            icp.wait()
            t = f * E + e

            def half(h, c):
                base = h * (OROWS * 128)

                def row(r, c2):
                    p = base + r * 128
                    for u in range(128 // L):
                        idx = idx_v[pl.ds(p + u * L, L)]
                        out_v[r, pl.ds(u * L, L)] = plsc.load_gather(
                            vec_v, [idx]
                        )
                    return c2

                lax.fori_loop(0, OROWS, row, 0)
                pltpu.sync_copy(out_v, out_hbm.at[t, pl.ds(h * OROWS, OROWS), :])
                return c

            lax.fori_loop(0, B // (OROWS * 128), half, 0)
            return carry

        lax.fori_loop(0, F, task, 0)

    return gather_kernel(xT, tables_t)


BB = 128
KSUB = 8
NBLK = B // (BB * KSUB)
D_IN = F * E


def _ln_relu_t(h, g, b):
    m = jnp.mean(h, axis=0, keepdims=True)
    v = jnp.mean((h - m) ** 2, axis=0, keepdims=True)
    return jnp.maximum((h - m) / jnp.sqrt(v + EPS) * g + b, 0.0)


def _mlp_body(emb_ref, w1t_ref, b1_ref, g1_ref, be1_ref, w2t_ref, b2_ref,
              g2_ref, be2_ref, w3t_ref, b3_ref, g3_ref, be3_ref, wf_ref,
              bf_ref, out_ref):
    for k in range(KSUB):
        eb = emb_ref[:, k, :]
        h = _ln_relu_t(
            jnp.dot(w1t_ref[...], eb, preferred_element_type=jnp.float32)
            + b1_ref[...], g1_ref[...], be1_ref[...])
        h = _ln_relu_t(
            jnp.dot(w2t_ref[...], h, preferred_element_type=jnp.float32)
            + b2_ref[...], g2_ref[...], be2_ref[...])
        h = _ln_relu_t(
            jnp.dot(w3t_ref[...], h, preferred_element_type=jnp.float32)
            + b3_ref[...], g3_ref[...], be3_ref[...])
        logits = jnp.sum(h * wf_ref[...], axis=0) + bf_ref[0, 0]
        out_ref[0, k, :] = 1.0 / (1.0 + jnp.exp(-logits))


def _tc_mlp(emb3, W1, b1, g1, be1, W2, b2, g2, be2, W3, b3, g3, be3, Wf, bf):
    col = lambda a: a.reshape(-1, 1)
    full = lambda s: pl.BlockSpec(s, lambda i: (0,) * len(s))
    out = pl.pallas_call(
        _mlp_body,
        grid=(NBLK,),
        in_specs=[
            pl.BlockSpec((D_IN, KSUB, BB), lambda i: (0, i, 0)),
            full((64, D_IN)),
            full((64, 1)), full((64, 1)), full((64, 1)),
            full((32, 64)), full((32, 1)), full((32, 1)), full((32, 1)),
            full((16, 32)), full((16, 1)), full((16, 1)), full((16, 1)),
            full((16, 1)), full((1, 1)),
        ],
        out_specs=pl.BlockSpec((1, KSUB, BB), lambda i: (i, 0, 0)),
        out_shape=jax.ShapeDtypeStruct((NBLK, KSUB, BB), jnp.float32),
    )(emb3, W1.T, col(b1), col(g1), col(be1), W2.T, col(b2), col(g2), col(be2),
      W3.T, col(b3), col(g3), col(be3), Wf, bf.reshape(1, 1))
    return out.reshape(B)


def kernel(x, tables, W1, b1, g1, be1, W2, b2, g2, be2, W3, b3, g3, be3, Wf, bf):
    xT = x.T
    tables_t = tables.transpose(0, 2, 1)
    emb3 = _sc_gather(xT, tables_t)
    return _tc_mlp(emb3, W1, b1, g1, be1, W2, b2, g2, be2, W3, b3, g3, be3, Wf, bf)


# speedup vs baseline: 27.8291x; 1.0134x over previous
import functools

import jax
import jax.numpy as jnp
from jax import lax
from jax.experimental import pallas as pl
from jax.experimental.pallas import tpu as pltpu
from jax.experimental.pallas import tpu_sc as plsc

F = 26
V = 100000
E = 32
B = 16384
EPS = 1e-5

NC, NS, L = 2, 16, 16
NW = NC * NS
HALF = B // 2
UNROLL = 4


OROWS = 32
VQ = V // 4


def _sc_gather(xT, tables_t):
    mesh = plsc.VectorSubcoreMesh(
        core_axis_name="c", subcore_axis_name="s", num_cores=NC, num_subcores=NS
    )

    @functools.partial(
        pl.kernel,
        out_type=jax.ShapeDtypeStruct((F * E, 128, 128), jnp.float32),
        mesh=mesh,
        scratch_types=[
            pltpu.VMEM((V,), jnp.float32),
            pltpu.VMEM((B,), jnp.int32),
            pltpu.VMEM((2, OROWS, 128), jnp.float32),
            pltpu.SemaphoreType.DMA,
            pltpu.SemaphoreType.DMA,
            pltpu.SemaphoreType.DMA,
            pltpu.SemaphoreType.DMA,
        ],
        compiler_params=pltpu.CompilerParams(
            use_tc_tiling_on_sc=True, needs_layout_passes=False
        ),
    )
    def gather_kernel(x_hbm, tab_hbm, out_hbm, vec_v, idx_v, out_v, vsem, isem,
                      osem0, osem1):
        e = lax.axis_index("s") * NC + lax.axis_index("c")
        osems = (osem0, osem1)
        ocps = [None, None]
        nq = B // (OROWS * 128)

        for f in range(F):
            vcp = pltpu.async_copy(tab_hbm.at[f, e], vec_v, vsem)
            icp = pltpu.async_copy(x_hbm.at[f], idx_v, isem)
            vcp.wait()
            icp.wait()
            t = f * E + e

            for q in range(nq):
                p = q % 2
                if ocps[p] is not None:
                    ocps[p].wait()
                base = q * (OROWS * 128)

                def row(r, c2, base=base, p=p):
                    pos = base + r * 128
                    for u in range(128 // L):
                        idx = idx_v[pl.ds(pos + u * L, L)]
                        out_v[p, r, pl.ds(u * L, L)] = plsc.load_gather(
                            vec_v, [idx]
                        )
                    return c2

                lax.fori_loop(0, OROWS, row, 0)
                ocps[p] = pltpu.async_copy(
                    out_v.at[p], out_hbm.at[t, pl.ds(q * OROWS, OROWS), :],
                    osems[p],
                )
        ocps[0].wait()
        ocps[1].wait()

    return gather_kernel(xT, tables_t)


BB = 128
KSUB = 8
NBLK = B // (BB * KSUB)
D_IN = F * E


def _ln_relu_t(h, g, b):
    m = jnp.mean(h, axis=0, keepdims=True)
    v = jnp.mean((h - m) ** 2, axis=0, keepdims=True)
    return jnp.maximum((h - m) / jnp.sqrt(v + EPS) * g + b, 0.0)


def _mlp_body(emb_ref, w1t_ref, b1_ref, g1_ref, be1_ref, w2t_ref, b2_ref,
              g2_ref, be2_ref, w3t_ref, b3_ref, g3_ref, be3_ref, wf_ref,
              bf_ref, out_ref):
    for k in range(KSUB):
        eb = emb_ref[:, k, :]
        h = _ln_relu_t(
            jnp.dot(w1t_ref[...], eb, preferred_element_type=jnp.float32)
            + b1_ref[...], g1_ref[...], be1_ref[...])
        h = _ln_relu_t(
            jnp.dot(w2t_ref[...], h, preferred_element_type=jnp.float32)
            + b2_ref[...], g2_ref[...], be2_ref[...])
        h = _ln_relu_t(
            jnp.dot(w3t_ref[...], h, preferred_element_type=jnp.float32)
            + b3_ref[...], g3_ref[...], be3_ref[...])
        logits = jnp.sum(h * wf_ref[...], axis=0) + bf_ref[0, 0]
        out_ref[0, k, :] = 1.0 / (1.0 + jnp.exp(-logits))


def _tc_mlp(emb3, W1, b1, g1, be1, W2, b2, g2, be2, W3, b3, g3, be3, Wf, bf):
    col = lambda a: a.reshape(-1, 1)
    full = lambda s: pl.BlockSpec(s, lambda i: (0,) * len(s))
    out = pl.pallas_call(
        _mlp_body,
        grid=(NBLK,),
        in_specs=[
            pl.BlockSpec((D_IN, KSUB, BB), lambda i: (0, i, 0)),
            full((64, D_IN)),
            full((64, 1)), full((64, 1)), full((64, 1)),
            full((32, 64)), full((32, 1)), full((32, 1)), full((32, 1)),
            full((16, 32)), full((16, 1)), full((16, 1)), full((16, 1)),
            full((16, 1)), full((1, 1)),
        ],
        out_specs=pl.BlockSpec((1, KSUB, BB), lambda i: (i, 0, 0)),
        out_shape=jax.ShapeDtypeStruct((NBLK, KSUB, BB), jnp.float32),
    )(emb3, W1.T, col(b1), col(g1), col(be1), W2.T, col(b2), col(g2), col(be2),
      W3.T, col(b3), col(g3), col(be3), Wf, bf.reshape(1, 1))
    return out.reshape(B)


def kernel(x, tables, W1, b1, g1, be1, W2, b2, g2, be2, W3, b3, g3, be3, Wf, bf):
    xT = x.T
    tables_t = tables.transpose(0, 2, 1)
    emb3 = _sc_gather(xT, tables_t)
    return _tc_mlp(emb3, W1, b1, g1, be1, W2, b2, g2, be2, W3, b3, g3, be3, Wf, bf)
